# Initial kernel scaffold; baseline (speedup 1.0000x reference)
import jax
import jax.numpy as jnp
from jax.experimental import pallas as pl


def kernel(x, edge_index, batch, params):
    raise NotImplementedError("write your pallas kernel here")


# speedup vs baseline: 9.0660x; 9.0660x over previous
import functools

import jax
import jax.numpy as jnp
from jax import lax
from jax.experimental import pallas as pl
from jax.experimental.pallas import tpu as pltpu
from jax.experimental.pallas import tpu_sc as plsc

N = 10000
E = 320000
G = 128
NC = 2
NS = 16
NW = NC * NS
EPT = E // NW
CH = 125
NCHUNK = EPT // CH
RPT = 632
N_PAD = RPT * NS


def _make_sc_agg(D):
  mesh = plsc.VectorSubcoreMesh(core_axis_name="c", subcore_axis_name="s")

  @functools.partial(
      pl.kernel,
      out_type=jax.ShapeDtypeStruct((NC, N_PAD, D), jnp.float32),
      mesh=mesh,
      compiler_params=pltpu.CompilerParams(use_tc_tiling_on_sc=False),
      scratch_types=[
          pltpu.VMEM((NCHUNK, CH), jnp.int32),
          pltpu.VMEM((NCHUNK, CH), jnp.int32),
          pltpu.VMEM((CH, D), jnp.float32),
          pltpu.SemaphoreType.DMA,
          pltpu.VMEM_SHARED((N_PAD, D), jnp.float32),
      ],
  )
  def k(h_hbm, src_hbm, dst_hbm, zero_hbm, out_hbm, src_v, dst_v, rows_v,
        sem, acc):
    cid = lax.axis_index("c")
    sid = lax.axis_index("s")
    tid = cid * NS + sid

    r0 = sid * RPT
    pltpu.sync_copy(zero_hbm, acc.at[pl.ds(r0, RPT)])
    pltpu.sync_copy(src_hbm.at[tid], src_v)
    pltpu.sync_copy(dst_hbm.at[tid], dst_v)
    plsc.subcore_barrier()

    def body(j, carry):
      pltpu.async_copy(h_hbm.at[src_v.at[j]], rows_v, sem).wait()
      pltpu.sync_copy(rows_v, acc.at[dst_v.at[j]], add=True)
      return carry

    lax.fori_loop(0, NCHUNK, body, 0)
    plsc.subcore_barrier()
    pltpu.sync_copy(acc.at[pl.ds(r0, RPT)], out_hbm.at[cid, pl.ds(r0, RPT)])

  return k


_sc_agg = {D: _make_sc_agg(D) for D in (128, 32, 64)}


def _gin_mlp_body(x_ref, p_ref, w1, b1, w2, b2, g, bt, o_ref):
  h = x_ref[...] + p_ref[0, :N] + p_ref[1, :N]
  h = jnp.maximum(
      jnp.dot(h, w1[...], preferred_element_type=jnp.float32) + b1[...], 0.0)
  h = jnp.dot(h, w2[...], preferred_element_type=jnp.float32) + b2[...]
  mu = jnp.mean(h, axis=0, keepdims=True)
  var = jnp.mean((h - mu) ** 2, axis=0, keepdims=True)
  h = (h - mu) * lax.rsqrt(var + 1e-5) * g[...] + bt[...]
  o_ref[...] = jnp.maximum(h, 0.0)


def _gin_mlp(x, p, w1, b1, w2, b2, g, bt):
  dout = w2.shape[1]
  return pl.pallas_call(
      _gin_mlp_body,
      out_shape=jax.ShapeDtypeStruct((N, dout), jnp.float32),
  )(x, p, w1, b1, w2, b2, g, bt)


def _final_body(x_ref, p_ref, w1, b1, w2, b2, g, bt, batch_ref,
                mw1, mb1, mw2, mb2, o_ref):
  h = x_ref[...] + p_ref[0, :N] + p_ref[1, :N]
  h = jnp.maximum(
      jnp.dot(h, w1[...], preferred_element_type=jnp.float32) + b1[...], 0.0)
  h = jnp.dot(h, w2[...], preferred_element_type=jnp.float32) + b2[...]
  mu = jnp.mean(h, axis=0, keepdims=True)
  var = jnp.mean((h - mu) ** 2, axis=0, keepdims=True)
  h = (h - mu) * lax.rsqrt(var + 1e-5) * g[...] + bt[...]
  h = jnp.maximum(h, 0.0)
  seg = (batch_ref[...] ==
         lax.broadcasted_iota(jnp.int32, (G, N), 0)).astype(jnp.float32)
  pooled = jnp.dot(seg, h, preferred_element_type=jnp.float32)
  hm = jnp.maximum(
      jnp.dot(pooled, mw1[...], preferred_element_type=jnp.float32)
      + mb1[...], 0.0)
  o_ref[...] = jnp.dot(hm, mw2[...], preferred_element_type=jnp.float32) \
      + mb2[...]


def _final(x, p, w1, b1, w2, b2, g, bt, batch2d, mw1, mb1, mw2, mb2):
  return pl.pallas_call(
      _final_body,
      out_shape=jax.ShapeDtypeStruct((G, 1), jnp.float32),
  )(x, p, w1, b1, w2, b2, g, bt, batch2d, mw1, mb1, mw2, mb2)


def kernel(x, edge_index, batch, params):
  p = params
  src3 = edge_index[0].reshape(NW, NCHUNK, CH)
  dst3 = edge_index[1].reshape(NW, NCHUNK, CH)
  batch2d = batch.reshape(1, N)
  row = lambda v: v.reshape(1, -1)

  z128 = jnp.zeros((RPT, 128), jnp.float32)
  z32 = jnp.zeros((RPT, 32), jnp.float32)
  z64 = jnp.zeros((RPT, 64), jnp.float32)

  agg1 = _sc_agg[128](x, src3, dst3, z128)
  h1 = _gin_mlp(x, agg1, p['c1_W1'], row(p['c1_b1']), p['c1_W2'],
                row(p['c1_b2']), row(p['c1_g']), row(p['c1_bt']))
  agg2 = _sc_agg[32](h1, src3, dst3, z32)
  h2 = _gin_mlp(h1, agg2, p['c2_W1'], row(p['c2_b1']), p['c2_W2'],
                row(p['c2_b2']), row(p['c2_g']), row(p['c2_bt']))
  agg3 = _sc_agg[64](h2, src3, dst3, z64)
  out = _final(h2, agg3, p['c3_W1'], row(p['c3_b1']), p['c3_W2'],
               row(p['c3_b2']), row(p['c3_g']), row(p['c3_bt']),
               batch2d, p['m_W1'], row(p['m_b1']), p['m_W2'], row(p['m_b2']))
  return out.squeeze(1)
